# baseline (speedup 1.0000x reference)
import functools

import jax
import jax.numpy as jnp
from jax import lax
from jax.experimental import pallas as pl
from jax.experimental.pallas import tpu as pltpu
from jax.experimental.pallas import tpu_sc as plsc

N_NODES = 10000
N_EDGES = 320000
D = 128
N_GRAPHS = 1024
BATCH = 128
N_LEMMAS = 1000

NC = 2
NS = 16
NPAD = 10240
STRIPE = NPAD // NS
CH = 128
NPH = 2
CPP = 40
NCHUNK = NPH * CPP
E_PAD = NC * NS * NCHUNK * CH


def _sc_edge_agg(x, src4, dst4, zrows):
    mesh = plsc.VectorSubcoreMesh(core_axis_name="c", subcore_axis_name="s")

    @functools.partial(
        pl.kernel,
        mesh=mesh,
        out_type=jax.ShapeDtypeStruct((NC, NPAD, D), jnp.float32),
        scratch_types=[
            pltpu.VMEM((CPP, CH), jnp.int32),
            pltpu.VMEM((CPP, CH), jnp.int32),
            pltpu.VMEM((CH, D), jnp.float32),
            pltpu.VMEM((CH, D), jnp.float32),
            pltpu.VMEM_SHARED((NPAD, D), jnp.float32),
            pltpu.SemaphoreType.DMA,
            pltpu.SemaphoreType.DMA,
        ],
    )
    def k(x_hbm, src_hbm, dst_hbm, z_hbm, out_hbm,
          src_l, dst_l, rows0, rows1, acc, sem0, sem1):
        cid = lax.axis_index("c")
        sid = lax.axis_index("s")
        wid = cid * NS + sid
        pltpu.sync_copy(z_hbm, acc.at[pl.ds(sid * STRIPE, STRIPE)])
        plsc.subcore_barrier()

        for ph in range(NPH):
            pltpu.sync_copy(src_hbm.at[wid, ph], src_l)
            pltpu.sync_copy(dst_hbm.at[wid, ph], dst_l)
            def body(it, carry):
                pltpu.async_copy(x_hbm.at[src_l.at[it]], rows0, sem0).wait()
                pltpu.sync_copy(rows0, acc.at[dst_l.at[it]], add=True)
                return carry

            lax.fori_loop(0, CPP, body, 0)

        plsc.subcore_barrier()
        pltpu.sync_copy(acc.at[pl.ds(sid * STRIPE, STRIPE)],
                        out_hbm.at[cid, pl.ds(sid * STRIPE, STRIPE)])

    return k(x, src4, dst4, zrows)


NBLK = 10
BLK = N_NODES // NBLK


def _tc_body(pref, xref, gref, bgref, wmsg, wself,
             wv1, bv1, wv2, bv2, wq1, bq1, wq2, bq2, wl1, wl2, bl,
             vf_ref, log_ref, gacc):
    i = pl.program_id(0)

    @pl.when(i == 0)
    def _():
        gacc[...] = jnp.zeros_like(gacc)

    xa = pref[0] + pref[1]
    state = jnp.maximum(
        jnp.dot(xa, wmsg[...], preferred_element_type=jnp.float32)
        + jnp.dot(xref[...], wself[...], preferred_element_type=jnp.float32),
        0.0)
    g = gref[0]
    oht = (g == lax.broadcasted_iota(jnp.int32, (N_GRAPHS, BLK), 0)
           ).astype(jnp.float32)
    gacc[...] += jnp.dot(oht, state, preferred_element_type=jnp.float32)

    @pl.when(i == NBLK - 1)
    def _():
        bg = bgref[0]
        ohb = (bg == lax.broadcasted_iota(jnp.int32, (BATCH, N_GRAPHS), 0)
               ).astype(jnp.float32)
        obj = jnp.dot(ohb, gacc[...], preferred_element_type=jnp.float32)
        v = jnp.maximum(
            jnp.dot(obj, wv1[...], preferred_element_type=jnp.float32)
            + bv1[...], 0.0)
        vf_ref[...] = jax.nn.sigmoid(
            jnp.dot(v, wv2[...], preferred_element_type=jnp.float32)
            + bv2[...])
        h = jnp.dot(
            jnp.maximum(
                jnp.dot(obj, wq1[...], preferred_element_type=jnp.float32)
                + bq1[...], 0.0),
            wq2[...], preferred_element_type=jnp.float32) + bq2[...]
        q1 = jnp.maximum(obj + h[:, :D], 0.0)
        q2 = jnp.maximum(h[:, D:], 0.0)
        log_ref[...] = (
            jnp.dot(q1, wl1[...], preferred_element_type=jnp.float32)
            + jnp.dot(q2, wl2[...], preferred_element_type=jnp.float32)
            + bl[...])


def kernel(x, edge_index, gnn_ind, batch_gnn_ind, W_msg, W_self,
           Wq1, bq1, Wq2, bq2, Wl, bl, Wv1, bv1, Wv2, bv2):
    src = edge_index[0].astype(jnp.int32)
    dst = edge_index[1].astype(jnp.int32)
    npad_e = E_PAD - N_EDGES
    src4 = jnp.concatenate(
        [src, jnp.zeros((npad_e,), jnp.int32)]).reshape(NC * NS, NPH, CPP, CH)
    dst4 = jnp.concatenate(
        [dst, jnp.full((npad_e,), NPAD - 1, jnp.int32)]).reshape(NC * NS, NPH, CPP, CH)
    zrows = jnp.zeros((STRIPE, D), jnp.float32)

    p = _sc_edge_agg(x, src4, dst4, zrows)

    gnn3 = gnn_ind.astype(jnp.int32).reshape(NBLK, 1, BLK)
    bgi3 = batch_gnn_ind.astype(jnp.int32).reshape(1, 1, N_GRAPHS)

    full = lambda s: pl.BlockSpec(s, lambda i: tuple(0 for _ in s))
    vf, logits = pl.pallas_call(
        _tc_body,
        grid=(NBLK,),
        in_specs=[
            pl.BlockSpec((NC, BLK, D), lambda i: (0, i, 0)),
            pl.BlockSpec((BLK, D), lambda i: (i, 0)),
            pl.BlockSpec((1, 1, BLK), lambda i: (i, 0, 0)),
            pl.BlockSpec((1, 1, N_GRAPHS), lambda i: (0, 0, 0)),
            full((D, D)), full((D, D)),
            full((D, D)), full((1, D)), full((D, 1)), full((1, 1)),
            full((D, 2 * D)), full((1, 2 * D)),
            full((2 * D, 2 * D)), full((1, 2 * D)),
            full((D, N_LEMMAS)), full((D, N_LEMMAS)), full((1, N_LEMMAS)),
        ],
        out_specs=[
            pl.BlockSpec((BATCH, 1), lambda i: (0, 0)),
            pl.BlockSpec((BATCH, N_LEMMAS), lambda i: (0, 0)),
        ],
        out_shape=[
            jax.ShapeDtypeStruct((BATCH, 1), jnp.float32),
            jax.ShapeDtypeStruct((BATCH, N_LEMMAS), jnp.float32),
        ],
        scratch_shapes=[pltpu.VMEM((N_GRAPHS, D), jnp.float32)],
    )(p, x, gnn3, bgi3, W_msg, W_self,
      Wv1[:D], bv1.reshape(1, D), Wv2, bv2.reshape(1, 1),
      Wq1[:D], bq1.reshape(1, 2 * D), Wq2, bq2.reshape(1, 2 * D),
      Wl[:D], Wl[D:], bl.reshape(1, N_LEMMAS))

    return jnp.concatenate([vf, logits], axis=1)


# speedup vs baseline: 1.1055x; 1.1055x over previous
import functools

import jax
import jax.numpy as jnp
from jax import lax
from jax.experimental import pallas as pl
from jax.experimental.pallas import tpu as pltpu
from jax.experimental.pallas import tpu_sc as plsc

N_NODES = 10000
N_EDGES = 320000
D = 128
N_GRAPHS = 1024
BATCH = 128
N_LEMMAS = 1000

NC = 2
NS = 16
NPAD = 10240
STRIPE = NPAD // NS
CH = 128
NCHUNK = 80
E_PAD = NC * NS * NCHUNK * CH


def _sc_edge_agg(x, src2, dst2, zrows):
    mesh = plsc.VectorSubcoreMesh(core_axis_name="c", subcore_axis_name="s")

    @functools.partial(
        pl.kernel,
        mesh=mesh,
        out_type=jax.ShapeDtypeStruct((NC, NPAD, D), jnp.float32),
        scratch_types=[
            pltpu.VMEM((CH,), jnp.int32),
            pltpu.VMEM((CH,), jnp.int32),
            pltpu.VMEM((CH,), jnp.int32),
            pltpu.VMEM((CH,), jnp.int32),
            pltpu.VMEM((CH, D), jnp.float32),
            pltpu.VMEM((CH, D), jnp.float32),
            pltpu.VMEM_SHARED((NPAD, D), jnp.float32),
            pltpu.SemaphoreType.DMA,
            pltpu.SemaphoreType.DMA,
            pltpu.SemaphoreType.DMA,
            pltpu.SemaphoreType.DMA,
        ],
    )
    def k(x_hbm, src_hbm, dst_hbm, z_hbm, out_hbm,
          srcv0, dstv0, srcv1, dstv1, rows0, rows1, acc,
          semg0, semg1, semi0, semi1):
        cid = lax.axis_index("c")
        sid = lax.axis_index("s")
        wid = cid * NS + sid
        pltpu.sync_copy(z_hbm, acc.at[pl.ds(sid * STRIPE, STRIPE)])
        plsc.subcore_barrier()

        def idx_start(j, sv, dv, sem):
            pltpu.async_copy(src_hbm.at[wid, pl.ds(j * CH, CH)], sv, sem)
            pltpu.async_copy(dst_hbm.at[wid, pl.ds(j * CH, CH)], dv, sem)

        def idx_wait(j, sv, dv, sem):
            pltpu.make_async_copy(src_hbm.at[wid, pl.ds(j * CH, CH)], sv, sem).wait()
            pltpu.make_async_copy(dst_hbm.at[wid, pl.ds(j * CH, CH)], dv, sem).wait()

        idx_start(0, srcv0, dstv0, semi0)
        idx_start(1, srcv1, dstv1, semi1)
        idx_wait(0, srcv0, dstv0, semi0)
        pltpu.async_copy(x_hbm.at[srcv0], rows0, semg0)
        idx_wait(1, srcv1, dstv1, semi1)

        def body(it, carry):
            j0 = it * 2
            pltpu.async_copy(x_hbm.at[srcv1], rows1, semg1)
            pltpu.make_async_copy(x_hbm.at[srcv0], rows0, semg0).wait()
            pltpu.sync_copy(rows0, acc.at[dstv0], add=True)

            @pl.when(j0 + 2 < NCHUNK)
            def _():
                idx_start(j0 + 2, srcv0, dstv0, semi0)
                idx_wait(j0 + 2, srcv0, dstv0, semi0)
                pltpu.async_copy(x_hbm.at[srcv0], rows0, semg0)

            pltpu.make_async_copy(x_hbm.at[srcv1], rows1, semg1).wait()
            pltpu.sync_copy(rows1, acc.at[dstv1], add=True)

            @pl.when(j0 + 3 < NCHUNK)
            def _():
                idx_start(j0 + 3, srcv1, dstv1, semi1)
                idx_wait(j0 + 3, srcv1, dstv1, semi1)

            return carry

        lax.fori_loop(0, NCHUNK // 2, body, 0)

        plsc.subcore_barrier()
        pltpu.sync_copy(acc.at[pl.ds(sid * STRIPE, STRIPE)],
                        out_hbm.at[cid, pl.ds(sid * STRIPE, STRIPE)])

    return k(x, src2, dst2, zrows)


NBLK = 10
BLK = N_NODES // NBLK


def _tc_body(pref, xref, gref, bgref, wmsg, wself,
             wv1, bv1, wv2, bv2, wq1, bq1, wq2, bq2, wl1, wl2, bl,
             vf_ref, log_ref, gacc):
    i = pl.program_id(0)

    @pl.when(i == 0)
    def _():
        gacc[...] = jnp.zeros_like(gacc)

    xa = pref[0] + pref[1]
    state = jnp.maximum(
        jnp.dot(xa, wmsg[...], preferred_element_type=jnp.float32)
        + jnp.dot(xref[...], wself[...], preferred_element_type=jnp.float32),
        0.0)
    g = gref[0]
    oht = (g == lax.broadcasted_iota(jnp.int32, (N_GRAPHS, BLK), 0)
           ).astype(jnp.float32)
    gacc[...] += jnp.dot(oht, state, preferred_element_type=jnp.float32)

    @pl.when(i == NBLK - 1)
    def _():
        bg = bgref[0]
        ohb = (bg == lax.broadcasted_iota(jnp.int32, (BATCH, N_GRAPHS), 0)
               ).astype(jnp.float32)
        obj = jnp.dot(ohb, gacc[...], preferred_element_type=jnp.float32)
        v = jnp.maximum(
            jnp.dot(obj, wv1[...], preferred_element_type=jnp.float32)
            + bv1[...], 0.0)
        vf_ref[...] = jax.nn.sigmoid(
            jnp.dot(v, wv2[...], preferred_element_type=jnp.float32)
            + bv2[...])
        h = jnp.dot(
            jnp.maximum(
                jnp.dot(obj, wq1[...], preferred_element_type=jnp.float32)
                + bq1[...], 0.0),
            wq2[...], preferred_element_type=jnp.float32) + bq2[...]
        q1 = jnp.maximum(obj + h[:, :D], 0.0)
        q2 = jnp.maximum(h[:, D:], 0.0)
        log_ref[...] = (
            jnp.dot(q1, wl1[...], preferred_element_type=jnp.float32)
            + jnp.dot(q2, wl2[...], preferred_element_type=jnp.float32)
            + bl[...])


def kernel(x, edge_index, gnn_ind, batch_gnn_ind, W_msg, W_self,
           Wq1, bq1, Wq2, bq2, Wl, bl, Wv1, bv1, Wv2, bv2):
    src = edge_index[0].astype(jnp.int32)
    dst = edge_index[1].astype(jnp.int32)
    npad_e = E_PAD - N_EDGES
    src2 = jnp.concatenate(
        [src, jnp.zeros((npad_e,), jnp.int32)]).reshape(NC * NS, NCHUNK * CH)
    dst2 = jnp.concatenate(
        [dst, jnp.full((npad_e,), NPAD - 1, jnp.int32)]).reshape(NC * NS, NCHUNK * CH)
    zrows = jnp.zeros((STRIPE, D), jnp.float32)

    p = _sc_edge_agg(x, src2, dst2, zrows)

    gnn3 = gnn_ind.astype(jnp.int32).reshape(NBLK, 1, BLK)
    bgi3 = batch_gnn_ind.astype(jnp.int32).reshape(1, 1, N_GRAPHS)

    full = lambda s: pl.BlockSpec(s, lambda i: tuple(0 for _ in s))
    vf, logits = pl.pallas_call(
        _tc_body,
        grid=(NBLK,),
        in_specs=[
            pl.BlockSpec((NC, BLK, D), lambda i: (0, i, 0)),
            pl.BlockSpec((BLK, D), lambda i: (i, 0)),
            pl.BlockSpec((1, 1, BLK), lambda i: (i, 0, 0)),
            pl.BlockSpec((1, 1, N_GRAPHS), lambda i: (0, 0, 0)),
            full((D, D)), full((D, D)),
            full((D, D)), full((1, D)), full((D, 1)), full((1, 1)),
            full((D, 2 * D)), full((1, 2 * D)),
            full((2 * D, 2 * D)), full((1, 2 * D)),
            full((D, N_LEMMAS)), full((D, N_LEMMAS)), full((1, N_LEMMAS)),
        ],
        out_specs=[
            pl.BlockSpec((BATCH, 1), lambda i: (0, 0)),
            pl.BlockSpec((BATCH, N_LEMMAS), lambda i: (0, 0)),
        ],
        out_shape=[
            jax.ShapeDtypeStruct((BATCH, 1), jnp.float32),
            jax.ShapeDtypeStruct((BATCH, N_LEMMAS), jnp.float32),
        ],
        scratch_shapes=[pltpu.VMEM((N_GRAPHS, D), jnp.float32)],
    )(p, x, gnn3, bgi3, W_msg, W_self,
      Wv1[:D], bv1.reshape(1, D), Wv2, bv2.reshape(1, 1),
      Wq1[:D], bq1.reshape(1, 2 * D), Wq2, bq2.reshape(1, 2 * D),
      Wl[:D], Wl[D:], bl.reshape(1, N_LEMMAS))

    return jnp.concatenate([vf, logits], axis=1)


# speedup vs baseline: 1.1069x; 1.0013x over previous
import functools

import jax
import jax.numpy as jnp
from jax import lax
from jax.experimental import pallas as pl
from jax.experimental.pallas import tpu as pltpu
from jax.experimental.pallas import tpu_sc as plsc

N_NODES = 10000
N_EDGES = 320000
D = 128
N_GRAPHS = 1024
BATCH = 128
N_LEMMAS = 1000

NC = 2
NS = 16
NPAD = 10240
STRIPE = NPAD // NS
CH = 128
NCHUNK = 80
E_PAD = NC * NS * NCHUNK * CH


def _sc_edge_agg(x, src2, dst2, zrows):
    mesh = plsc.VectorSubcoreMesh(core_axis_name="c", subcore_axis_name="s")

    @functools.partial(
        pl.kernel,
        mesh=mesh,
        out_type=jax.ShapeDtypeStruct((NC, NPAD, D), jnp.float32),
        scratch_types=[
            pltpu.VMEM((CH,), jnp.int32),
            pltpu.VMEM((CH,), jnp.int32),
            pltpu.VMEM((CH,), jnp.int32),
            pltpu.VMEM((CH,), jnp.int32),
            pltpu.VMEM((CH, D), jnp.float32),
            pltpu.VMEM((CH, D), jnp.float32),
            pltpu.VMEM_SHARED((NPAD, D), jnp.float32),
            pltpu.SemaphoreType.DMA,
            pltpu.SemaphoreType.DMA,
            pltpu.SemaphoreType.DMA,
            pltpu.SemaphoreType.DMA,
        ],
    )
    def k(x_hbm, src_hbm, dst_hbm, z_hbm, out_hbm,
          srcv0, dstv0, srcv1, dstv1, rows0, rows1, acc,
          semg0, semg1, semi0, semi1):
        cid = lax.axis_index("c")
        sid = lax.axis_index("s")
        wid = cid * NS + sid
        pltpu.sync_copy(z_hbm, acc.at[pl.ds(sid * STRIPE, STRIPE)])
        plsc.subcore_barrier()

        def idx_start(j, sv, dv, sem):
            pltpu.async_copy(src_hbm.at[wid, pl.ds(j * CH, CH)], sv, sem)
            pltpu.async_copy(dst_hbm.at[wid, pl.ds(j * CH, CH)], dv, sem)

        def idx_wait(j, sv, dv, sem):
            pltpu.make_async_copy(src_hbm.at[wid, pl.ds(j * CH, CH)], sv, sem).wait()
            pltpu.make_async_copy(dst_hbm.at[wid, pl.ds(j * CH, CH)], dv, sem).wait()

        idx_start(0, srcv0, dstv0, semi0)
        idx_start(1, srcv1, dstv1, semi1)
        idx_wait(0, srcv0, dstv0, semi0)
        pltpu.async_copy(x_hbm.at[srcv0], rows0, semg0)
        idx_wait(1, srcv1, dstv1, semi1)

        def body(it, carry):
            j0 = it * 2
            pltpu.async_copy(x_hbm.at[srcv1], rows1, semg1)
            pltpu.make_async_copy(x_hbm.at[srcv0], rows0, semg0).wait()
            pltpu.sync_copy(rows0, acc.at[dstv0], add=True)

            @pl.when(j0 + 2 < NCHUNK)
            def _():
                idx_start(j0 + 2, srcv0, dstv0, semi0)
                idx_wait(j0 + 2, srcv0, dstv0, semi0)
                pltpu.async_copy(x_hbm.at[srcv0], rows0, semg0)

            pltpu.make_async_copy(x_hbm.at[srcv1], rows1, semg1).wait()
            pltpu.sync_copy(rows1, acc.at[dstv1], add=True)

            @pl.when(j0 + 3 < NCHUNK)
            def _():
                idx_start(j0 + 3, srcv1, dstv1, semi1)
                idx_wait(j0 + 3, srcv1, dstv1, semi1)

            return carry

        lax.fori_loop(0, NCHUNK // 2, body, 0)

        plsc.subcore_barrier()
        pltpu.sync_copy(acc.at[pl.ds(sid * STRIPE, STRIPE)],
                        out_hbm.at[cid, pl.ds(sid * STRIPE, STRIPE)])

    return k(x, src2, dst2, zrows)


NBLK = 10
BLK = N_NODES // NBLK


def _tc_body(pref, xref, gref, bgref, wmsg, wself,
             wv1, bv1, wv2, bv2, wq1, bq1, wq2, bq2, wl1, wl2, bl,
             vf_ref, log_ref, gacc):
    i = pl.program_id(0)

    @pl.when(i == 0)
    def _():
        gacc[...] = jnp.zeros_like(gacc)

    xa = pref[0] + pref[1]
    state = jnp.maximum(
        jnp.dot(xa, wmsg[...], preferred_element_type=jnp.float32)
        + jnp.dot(xref[...], wself[...], preferred_element_type=jnp.float32),
        0.0)
    g = gref[0]
    oht = (g == lax.broadcasted_iota(jnp.int32, (N_GRAPHS, BLK), 0)
           ).astype(jnp.float32)
    gacc[...] += jnp.dot(oht, state, preferred_element_type=jnp.float32)

    @pl.when(i == NBLK - 1)
    def _():
        bg = bgref[0]
        ohb = (bg == lax.broadcasted_iota(jnp.int32, (BATCH, N_GRAPHS), 0)
               ).astype(jnp.float32)
        obj = jnp.dot(ohb, gacc[...], preferred_element_type=jnp.float32)
        v = jnp.maximum(
            jnp.dot(obj, wv1[...], preferred_element_type=jnp.float32)
            + bv1[...], 0.0)
        vf_ref[...] = jax.nn.sigmoid(
            jnp.dot(v, wv2[...], preferred_element_type=jnp.float32)
            + bv2[...])
        h = jnp.dot(
            jnp.maximum(
                jnp.dot(obj, wq1[...], preferred_element_type=jnp.float32)
                + bq1[...], 0.0),
            wq2[...], preferred_element_type=jnp.float32) + bq2[...]
        q1 = jnp.maximum(obj + h[:, :D], 0.0)
        q2 = jnp.maximum(h[:, D:], 0.0)
        log_ref[...] = (
            jnp.dot(q1, wl1[...], preferred_element_type=jnp.float32)
            + jnp.dot(q2, wl2[...], preferred_element_type=jnp.float32)
            + bl[...])


def kernel(x, edge_index, gnn_ind, batch_gnn_ind, W_msg, W_self,
           Wq1, bq1, Wq2, bq2, Wl, bl, Wv1, bv1, Wv2, bv2):
    src = edge_index[0].astype(jnp.int32)
    dst = edge_index[1].astype(jnp.int32)
    npad_e = E_PAD - N_EDGES
    src2 = jnp.concatenate(
        [src, jnp.zeros((npad_e,), jnp.int32)]).reshape(NC * NS, NCHUNK * CH)
    junk = N_NODES + jnp.arange(npad_e, dtype=jnp.int32) % (NPAD - N_NODES)
    dst2 = jnp.concatenate([dst, junk]).reshape(NC * NS, NCHUNK * CH)
    zrows = jnp.zeros((STRIPE, D), jnp.float32)

    p = _sc_edge_agg(x, src2, dst2, zrows)

    gnn3 = gnn_ind.astype(jnp.int32).reshape(NBLK, 1, BLK)
    bgi3 = batch_gnn_ind.astype(jnp.int32).reshape(1, 1, N_GRAPHS)

    full = lambda s: pl.BlockSpec(s, lambda i: tuple(0 for _ in s))
    vf, logits = pl.pallas_call(
        _tc_body,
        grid=(NBLK,),
        in_specs=[
            pl.BlockSpec((NC, BLK, D), lambda i: (0, i, 0)),
            pl.BlockSpec((BLK, D), lambda i: (i, 0)),
            pl.BlockSpec((1, 1, BLK), lambda i: (i, 0, 0)),
            pl.BlockSpec((1, 1, N_GRAPHS), lambda i: (0, 0, 0)),
            full((D, D)), full((D, D)),
            full((D, D)), full((1, D)), full((D, 1)), full((1, 1)),
            full((D, 2 * D)), full((1, 2 * D)),
            full((2 * D, 2 * D)), full((1, 2 * D)),
            full((D, N_LEMMAS)), full((D, N_LEMMAS)), full((1, N_LEMMAS)),
        ],
        out_specs=[
            pl.BlockSpec((BATCH, 1), lambda i: (0, 0)),
            pl.BlockSpec((BATCH, N_LEMMAS), lambda i: (0, 0)),
        ],
        out_shape=[
            jax.ShapeDtypeStruct((BATCH, 1), jnp.float32),
            jax.ShapeDtypeStruct((BATCH, N_LEMMAS), jnp.float32),
        ],
        scratch_shapes=[pltpu.VMEM((N_GRAPHS, D), jnp.float32)],
    )(p, x, gnn3, bgi3, W_msg, W_self,
      Wv1[:D], bv1.reshape(1, D), Wv2, bv2.reshape(1, 1),
      Wq1[:D], bq1.reshape(1, 2 * D), Wq2, bq2.reshape(1, 2 * D),
      Wl[:D], Wl[D:], bl.reshape(1, N_LEMMAS))

    return jnp.concatenate([vf, logits], axis=1)


# speedup vs baseline: 3.2939x; 2.9756x over previous
import functools

import jax
import jax.numpy as jnp
from jax import lax
from jax.experimental import pallas as pl
from jax.experimental.pallas import tpu as pltpu
from jax.experimental.pallas import tpu_sc as plsc

N_NODES = 10000
N_EDGES = 320000
D = 128
N_GRAPHS = 1024
BATCH = 128
N_LEMMAS = 1000

NC = 2
NS = 16
NPAD = 10240
STRIPE = NPAD // NS
CH = 128
NCHUNK = 80
E_PAD = NC * NS * NCHUNK * CH


def _sc_edge_agg(x, src2, dst2, zrows):
    mesh = plsc.VectorSubcoreMesh(core_axis_name="c", subcore_axis_name="s")

    @functools.partial(
        pl.kernel,
        mesh=mesh,
        out_type=jax.ShapeDtypeStruct((NC, NPAD, D), jnp.float32),
        scratch_types=[
            pltpu.VMEM((CH,), jnp.int32),
            pltpu.VMEM((CH,), jnp.int32),
            pltpu.VMEM((CH,), jnp.int32),
            pltpu.VMEM((CH,), jnp.int32),
            pltpu.VMEM((CH, D), jnp.float32),
            pltpu.VMEM((CH, D), jnp.float32),
            pltpu.VMEM_SHARED((NPAD, D), jnp.float32),
            pltpu.SemaphoreType.DMA,
            pltpu.SemaphoreType.DMA,
            pltpu.SemaphoreType.DMA,
            pltpu.SemaphoreType.DMA,
        ],
    )
    def k(x_hbm, src_hbm, dst_hbm, z_hbm, out_hbm,
          srcv0, dstv0, srcv1, dstv1, rows0, rows1, acc,
          semg0, semg1, semi0, semi1):
        cid = lax.axis_index("c")
        sid = lax.axis_index("s")
        wid = cid * NS + sid
        pltpu.sync_copy(z_hbm, acc.at[pl.ds(sid * STRIPE, STRIPE)])
        plsc.subcore_barrier()

        def idx_start(j, sv, dv, sem):
            pltpu.async_copy(src_hbm.at[wid, pl.ds(j * CH, CH)], sv, sem)
            pltpu.async_copy(dst_hbm.at[wid, pl.ds(j * CH, CH)], dv, sem)

        def idx_wait(j, sv, dv, sem):
            pltpu.make_async_copy(src_hbm.at[wid, pl.ds(j * CH, CH)], sv, sem).wait()
            pltpu.make_async_copy(dst_hbm.at[wid, pl.ds(j * CH, CH)], dv, sem).wait()

        idx_start(0, srcv0, dstv0, semi0)
        idx_start(1, srcv1, dstv1, semi1)
        idx_wait(0, srcv0, dstv0, semi0)
        pltpu.async_copy(x_hbm.at[srcv0], rows0, semg0)
        idx_wait(1, srcv1, dstv1, semi1)

        def body(it, carry):
            j0 = it * 2
            pltpu.async_copy(x_hbm.at[srcv1], rows1, semg1)
            pltpu.make_async_copy(x_hbm.at[srcv0], rows0, semg0).wait()
            pltpu.sync_copy(rows0, acc.at[dstv0], add=True)

            @pl.when(j0 + 2 < NCHUNK)
            def _():
                idx_start(j0 + 2, srcv0, dstv0, semi0)
                idx_wait(j0 + 2, srcv0, dstv0, semi0)
                pltpu.async_copy(x_hbm.at[srcv0], rows0, semg0)

            pltpu.make_async_copy(x_hbm.at[srcv1], rows1, semg1).wait()
            pltpu.sync_copy(rows1, acc.at[dstv1], add=True)

            @pl.when(j0 + 3 < NCHUNK)
            def _():
                idx_start(j0 + 3, srcv1, dstv1, semi1)
                idx_wait(j0 + 3, srcv1, dstv1, semi1)

            return carry

        lax.fori_loop(0, NCHUNK // 2, body, 0)

        plsc.subcore_barrier()
        pltpu.sync_copy(acc.at[pl.ds(sid * STRIPE, STRIPE)],
                        out_hbm.at[cid, pl.ds(sid * STRIPE, STRIPE)])

    return k(x, src2, dst2, zrows)


NBLK = 10
BLK = N_NODES // NBLK


def _tc_body(pref, xref, gref, bgref, wmsg, wself,
             wv1, bv1, wv2, bv2, wq1, bq1, wq2, bq2, wl1, wl2, bl,
             vf_ref, log_ref, gacc):
    i = pl.program_id(0)

    @pl.when(i == 0)
    def _():
        gacc[...] = jnp.zeros_like(gacc)

    xa = pref[0] + pref[1]
    state = jnp.maximum(
        jnp.dot(xa, wmsg[...], preferred_element_type=jnp.float32)
        + jnp.dot(xref[...], wself[...], preferred_element_type=jnp.float32),
        0.0)
    g = gref[0]
    oht = (g == lax.broadcasted_iota(jnp.int32, (N_GRAPHS, BLK), 0)
           ).astype(jnp.float32)
    gacc[...] += jnp.dot(oht, state, preferred_element_type=jnp.float32)

    @pl.when(i == NBLK - 1)
    def _():
        bg = bgref[0]
        ohb = (bg == lax.broadcasted_iota(jnp.int32, (BATCH, N_GRAPHS), 0)
               ).astype(jnp.float32)
        obj = jnp.dot(ohb, gacc[...], preferred_element_type=jnp.float32)
        v = jnp.maximum(
            jnp.dot(obj, wv1[...], preferred_element_type=jnp.float32)
            + bv1[...], 0.0)
        vf_ref[...] = jax.nn.sigmoid(
            jnp.dot(v, wv2[...], preferred_element_type=jnp.float32)
            + bv2[...])
        h = jnp.dot(
            jnp.maximum(
                jnp.dot(obj, wq1[...], preferred_element_type=jnp.float32)
                + bq1[...], 0.0),
            wq2[...], preferred_element_type=jnp.float32) + bq2[...]
        q1 = jnp.maximum(obj + h[:, :D], 0.0)
        q2 = jnp.maximum(h[:, D:], 0.0)
        log_ref[...] = (
            jnp.dot(q1, wl1[...], preferred_element_type=jnp.float32)
            + jnp.dot(q2, wl2[...], preferred_element_type=jnp.float32)
            + bl[...])


def kernel(x, edge_index, gnn_ind, batch_gnn_ind, W_msg, W_self,
           Wq1, bq1, Wq2, bq2, Wl, bl, Wv1, bv1, Wv2, bv2):
    src = edge_index[0].astype(jnp.int32)
    dst = edge_index[1].astype(jnp.int32)
    npad_e = E_PAD - N_EDGES
    padsrc = jnp.arange(npad_e, dtype=jnp.int32) % N_NODES
    src2 = jnp.concatenate([src, padsrc]).reshape(NC * NS, NCHUNK * CH)
    junk = N_NODES + jnp.arange(npad_e, dtype=jnp.int32) % (NPAD - N_NODES)
    dst2 = jnp.concatenate([dst, junk]).reshape(NC * NS, NCHUNK * CH)
    zrows = jnp.zeros((STRIPE, D), jnp.float32)

    p = _sc_edge_agg(x, src2, dst2, zrows)

    gnn3 = gnn_ind.astype(jnp.int32).reshape(NBLK, 1, BLK)
    bgi3 = batch_gnn_ind.astype(jnp.int32).reshape(1, 1, N_GRAPHS)

    full = lambda s: pl.BlockSpec(s, lambda i: tuple(0 for _ in s))
    vf, logits = pl.pallas_call(
        _tc_body,
        grid=(NBLK,),
        in_specs=[
            pl.BlockSpec((NC, BLK, D), lambda i: (0, i, 0)),
            pl.BlockSpec((BLK, D), lambda i: (i, 0)),
            pl.BlockSpec((1, 1, BLK), lambda i: (i, 0, 0)),
            pl.BlockSpec((1, 1, N_GRAPHS), lambda i: (0, 0, 0)),
            full((D, D)), full((D, D)),
            full((D, D)), full((1, D)), full((D, 1)), full((1, 1)),
            full((D, 2 * D)), full((1, 2 * D)),
            full((2 * D, 2 * D)), full((1, 2 * D)),
            full((D, N_LEMMAS)), full((D, N_LEMMAS)), full((1, N_LEMMAS)),
        ],
        out_specs=[
            pl.BlockSpec((BATCH, 1), lambda i: (0, 0)),
            pl.BlockSpec((BATCH, N_LEMMAS), lambda i: (0, 0)),
        ],
        out_shape=[
            jax.ShapeDtypeStruct((BATCH, 1), jnp.float32),
            jax.ShapeDtypeStruct((BATCH, N_LEMMAS), jnp.float32),
        ],
        scratch_shapes=[pltpu.VMEM((N_GRAPHS, D), jnp.float32)],
    )(p, x, gnn3, bgi3, W_msg, W_self,
      Wv1[:D], bv1.reshape(1, D), Wv2, bv2.reshape(1, 1),
      Wq1[:D], bq1.reshape(1, 2 * D), Wq2, bq2.reshape(1, 2 * D),
      Wl[:D], Wl[D:], bl.reshape(1, N_LEMMAS))

    return jnp.concatenate([vf, logits], axis=1)
